# baseline (speedup 1.0000x reference)
import functools

import jax
import jax.numpy as jnp
from jax import lax
from jax.experimental import pallas as pl
from jax.experimental.pallas import tpu as pltpu
from jax.experimental.pallas import tpu_sc as plsc

VOCAB = 1000000
EMBED = 64
NUM_CLASSES = 16
BATCH = 4096
SEQ = 200

NC = 2
NS = 16
NW = NC * NS
B_PER_W = BATCH // NW
LANES = 16
QV = EMBED // LANES
UNROLL = 8
SEQ_PAD = 256
IDX_PER_W = B_PER_W * SEQ_PAD


def _pool_body(x_hbm, table_hbm, out_hbm, idx_v, rows_v, pooled_v, sem0, sem1):
    wid = lax.axis_index("s") * NC + lax.axis_index("c")
    base = wid * B_PER_W

    pltpu.sync_copy(x_hbm.at[pl.ds(wid * IDX_PER_W, IDX_PER_W)], idx_v)

    sems = (sem0, sem1)

    def idx_slice(i):
        off = pl.multiple_of(i * SEQ_PAD, SEQ_PAD)
        return idx_v.at[pl.ds(off, SEQ)]

    pltpu.async_copy(table_hbm.at[idx_slice(0)], rows_v.at[0], sem0)

    neg_inf = jnp.full((LANES,), -jnp.inf, dtype=jnp.float32)

    def outer(g, carry):
        for b in range(2):
            i = g * 2 + b
            pltpu.make_async_copy(
                table_hbm.at[idx_slice(0)], rows_v.at[b], sems[b]
            ).wait()

            @pl.when(i + 1 < B_PER_W)
            def _():
                pltpu.async_copy(
                    table_hbm.at[idx_slice(i + 1)],
                    rows_v.at[1 - b],
                    sems[1 - b],
                )

            def red(t, accs):
                a = list(accs)
                j0 = t * UNROLL
                for u in range(UNROLL):
                    for q in range(QV):
                        a[q] = jnp.maximum(
                            a[q], rows_v[b, j0 + u, pl.ds(q * LANES, LANES)]
                        )
                return tuple(a)

            accs = lax.fori_loop(
                0, SEQ // UNROLL, red, (neg_inf,) * QV, unroll=False
            )
            for q in range(QV):
                pooled_v[i, pl.ds(q * LANES, LANES)] = accs[q]
        return carry

    lax.fori_loop(0, B_PER_W // 2, outer, 0, unroll=False)

    pltpu.sync_copy(pooled_v, out_hbm.at[pl.ds(base, B_PER_W)])


_pool = functools.partial(
    pl.kernel,
    out_type=jax.ShapeDtypeStruct((BATCH, EMBED), jnp.float32),
    mesh=plsc.VectorSubcoreMesh(core_axis_name="c", subcore_axis_name="s"),
    scratch_types=[
        pltpu.VMEM((IDX_PER_W,), jnp.int32),
        pltpu.VMEM((2, SEQ, EMBED), jnp.float32),
        pltpu.VMEM((B_PER_W, EMBED), jnp.float32),
        pltpu.SemaphoreType.DMA,
        pltpu.SemaphoreType.DMA,
    ],
    compiler_params=pltpu.CompilerParams(use_tc_tiling_on_sc=False),
)(_pool_body)


def _mlp_body(p_ref, w1_ref, b1_ref, w2_ref, b2_ref, o_ref):
    h = jnp.maximum(
        jnp.dot(p_ref[...], w1_ref[...], preferred_element_type=jnp.float32)
        + b1_ref[...],
        0.0,
    )
    o_ref[...] = (
        jnp.dot(h, w2_ref[...], preferred_element_type=jnp.float32)
        + b2_ref[...]
    )


def kernel(x, table, W1, b1, W2, b2):
    xi = jnp.pad(x.astype(jnp.int32), ((0, 0), (0, SEQ_PAD - SEQ)))
    pooled = _pool(xi.reshape(-1), table)
    out = pl.pallas_call(
        _mlp_body,
        out_shape=jax.ShapeDtypeStruct((BATCH, NUM_CLASSES), jnp.float32),
    )(pooled, W1, b1.reshape(1, EMBED), W2, b2.reshape(1, NUM_CLASSES))
    return out
